# baseline (speedup 1.0000x reference)
import jax
import jax.numpy as jnp
from jax import lax
from jax.experimental import pallas as pl
from jax.experimental.pallas import tpu as pltpu
from jax.experimental.pallas import tpu_sc as plsc

_N = 50000
_E = 320000
_D = 128
_H = 128
_O = 64

_PADN = 52224
_CHUNK = 8704
_NCH = 3
_NS = 16
_B = 128
_PER_TILE = 20480
_EPAD = _NS * _PER_TILE
_SEG = 4096
_NSEG = _PER_TILE // _SEG
_NBATCH = _SEG // _B
_ACC_SL = _CHUNK // _NS
_BCAST_DNUMS = lax.GatherDimensionNumbers(
    offset_dims=(), collapsed_slice_dims=(0,), start_index_map=(0,))


def _sc_messages(src, dst, w, table):
  mesh = plsc.VectorSubcoreMesh(core_axis_name="c", subcore_axis_name="s")

  def body(src_hbm, dst_hbm, w_hbm, table_hbm, msg_hbm, deg_hbm,
           seg_src, seg_dst, seg_w, dst_b, w_b, rows, rows2, zbuf, deg_v,
           acc_sh, deg_sh, sem_a, sem_b):
    c = lax.axis_index("c")
    s = lax.axis_index("s")
    ebase = s * _PER_TILE
    zi = jnp.zeros((16,), jnp.int32)
    zf = jnp.zeros((16,), jnp.float32)

    def zero_zbuf(i, carry):
      zbuf[pl.ds(i * 16, 16)] = zf
      return carry
    lax.fori_loop(0, _ACC_SL // 16, zero_zbuf, 0)

    def chunk_body(k, carry):
      lo = (_NCH * c + k) * _CHUNK

      def zero_rows(i, carry):
        for r in range(_D // 16):
          rows[i, pl.ds(r * 16, 16)] = zf
        return carry
      lax.fori_loop(0, _B, zero_rows, 0)
      r0 = s * _ACC_SL
      for i in range(_ACC_SL // _B):
        pltpu.sync_copy(rows, acc_sh.at[pl.ds(r0 + i * _B, _B)])
      rem = _ACC_SL - (_ACC_SL // _B) * _B
      if rem:
        pltpu.sync_copy(rows.at[pl.ds(0, rem)],
                        acc_sh.at[pl.ds(r0 + (_ACC_SL // _B) * _B, rem)])
      pltpu.sync_copy(zbuf, deg_sh.at[pl.ds(r0, _ACC_SL)])
      plsc.subcore_barrier()

      def seg_body(si, carry2):
        base = ebase + si * _SEG
        pltpu.sync_copy(src_hbm.at[pl.ds(base, _SEG)], seg_src)
        pltpu.sync_copy(dst_hbm.at[pl.ds(base, _SEG)], seg_dst)
        pltpu.sync_copy(w_hbm.at[pl.ds(base, _SEG)], seg_w)

        def gather(off, buf, sem):
          return pltpu.make_async_copy(
              table_hbm.at[seg_src.at[pl.ds(off, _B)]], buf, sem)

        def process(off, buf):
          for g in range(_B // 16):
            dv = seg_dst[pl.ds(off + g * 16, 16)]
            wv = seg_w[pl.ds(off + g * 16, 16)]
            m = (dv >= lo) & (dv < lo + _CHUNK) & (wv > 0.0)
            dst_b[pl.ds(g * 16, 16)] = jnp.where(m, dv - lo, _CHUNK)
            w_b[pl.ds(g * 16, 16)] = jnp.where(m, wv, zf)
          for g in range(_B // 16):
            wv = w_b[pl.ds(g * 16, 16)]
            for bb in range(16):
              b = g * 16 + bb
              wb = lax.gather(
                  wv, jnp.full((16, 1), bb, jnp.int32), _BCAST_DNUMS,
                  slice_sizes=(1,),
                  mode=lax.GatherScatterMode.PROMISE_IN_BOUNDS)
              for r in range(_D // 16):
                buf[b, pl.ds(r * 16, 16)] = buf[b, pl.ds(r * 16, 16)] * wb
          pltpu.sync_copy(buf, acc_sh.at[dst_b], add=True)
          pltpu.sync_copy(w_b, deg_sh.at[dst_b], add=True)

        gather(0, rows, sem_a).start()

        def pair(pi, carry):
          off_a = 2 * pi * _B
          off_b = off_a + _B
          gather(off_b, rows2, sem_b).start()
          gather(off_a, rows, sem_a).wait()
          process(off_a, rows)

          @pl.when(pi < _NBATCH // 2 - 1)
          def _next():
            gather(off_a + 2 * _B, rows, sem_a).start()

          gather(off_b, rows2, sem_b).wait()
          process(off_b, rows2)
          return carry
        lax.fori_loop(0, _NBATCH // 2, pair, 0)
        return carry2
      lax.fori_loop(0, _NSEG, seg_body, 0)

      plsc.subcore_barrier()
      pltpu.sync_copy(acc_sh.at[pl.ds(r0, _ACC_SL)],
                      msg_hbm.at[pl.ds(lo + r0, _ACC_SL)])
      pltpu.sync_copy(deg_sh.at[pl.ds(r0, _ACC_SL)], deg_v)
      pltpu.sync_copy(deg_v, deg_hbm.at[pl.ds(lo + r0, _ACC_SL)])
      return carry
    lax.fori_loop(0, _NCH, chunk_body, 0)

  run = pl.kernel(
      body,
      out_type=(jax.ShapeDtypeStruct((_PADN, _D), jnp.float32),
                jax.ShapeDtypeStruct((_PADN,), jnp.float32)),
      mesh=mesh,
      scratch_types=[
          pltpu.VMEM((_SEG,), jnp.int32),
          pltpu.VMEM((_SEG,), jnp.int32),
          pltpu.VMEM((_SEG,), jnp.float32),
          pltpu.VMEM((_B,), jnp.int32),
          pltpu.VMEM((_B,), jnp.float32),
          pltpu.VMEM((_B, _D), jnp.float32),
          pltpu.VMEM((_B, _D), jnp.float32),
          pltpu.VMEM((_ACC_SL,), jnp.float32),
          pltpu.VMEM((_ACC_SL,), jnp.float32),
          pltpu.VMEM_SHARED((_CHUNK + 16, _D), jnp.float32),
          pltpu.VMEM_SHARED((_CHUNK + 16,), jnp.float32),
          pltpu.SemaphoreType.DMA,
          pltpu.SemaphoreType.DMA,
      ],
  )
  return run(src, dst, w, table)


def _layernorm(x, g, b, eps=1e-5):
  mu = jnp.mean(x, axis=-1, keepdims=True)
  var = jnp.mean((x - mu) ** 2, axis=-1, keepdims=True)
  return (x - mu) / jnp.sqrt(var + eps) * g + b


_BR = 2000


def _mlp_body(tab_ref, msg_ref, deg_ref, w1a_ref, w1b_ref, b1_ref, g1_ref,
              be1_ref, w2_ref, b2_ref, g2_ref, be2_ref, out_ref):
  m = msg_ref[...] / (deg_ref[...] + 1e-8)
  h = jnp.dot(tab_ref[...], w1a_ref[...], preferred_element_type=jnp.float32)
  h = h + jnp.dot(m, w1b_ref[...], preferred_element_type=jnp.float32)
  h = h + b1_ref[...]
  h = _layernorm(h, g1_ref[...], be1_ref[...])
  h = jnp.maximum(h, 0.0)
  o = jnp.dot(h, w2_ref[...], preferred_element_type=jnp.float32) + b2_ref[...]
  out_ref[...] = _layernorm(o, g2_ref[...], be2_ref[...])


def _tc_mlp(table, msg, deg, W1, b1, g1, be1, W2, b2, g2, be2):
  full = lambda shape: pl.BlockSpec(shape, lambda i: (0, 0))
  return pl.pallas_call(
      _mlp_body,
      grid=(_N // _BR,),
      in_specs=[
          pl.BlockSpec((_BR, _D), lambda i: (i, 0)),
          pl.BlockSpec((_BR, _D), lambda i: (i, 0)),
          pl.BlockSpec((_BR, 1), lambda i: (i, 0)),
          full((_D, _H)), full((_D, _H)), full((1, _H)), full((1, _H)),
          full((1, _H)), full((_H, _O)), full((1, _O)), full((1, _O)),
          full((1, _O)),
      ],
      out_specs=pl.BlockSpec((_BR, _O), lambda i: (i, 0)),
      out_shape=jax.ShapeDtypeStruct((_N, _O), jnp.float32),
  )(table, msg, deg, W1[:_D], W1[_D:], b1[None], g1[None], be1[None],
    W2, b2[None], g2[None], be2[None])


def kernel(provider_code_edges, code_provider_edges, edge_weights,
           provider_table, code_table,
           pW1, pb1, pg1, pbe1, pW2, pb2, pg2, pbe2,
           cW1, cb1, cg1, cbe1, cW2, cb2, cg2, cbe2):
  pad = _EPAD - _E
  w = jnp.pad(edge_weights.astype(jnp.float32), (0, pad))
  cp_src = jnp.pad(code_provider_edges[0].astype(jnp.int32), (0, pad))
  cp_dst = jnp.pad(code_provider_edges[1].astype(jnp.int32), (0, pad))
  pc_src = jnp.pad(provider_code_edges[0].astype(jnp.int32), (0, pad))
  pc_dst = jnp.pad(provider_code_edges[1].astype(jnp.int32), (0, pad))

  prov_msg, prov_deg = _sc_messages(cp_src, cp_dst, w, code_table)
  code_msg, code_deg = _sc_messages(pc_src, pc_dst, w, provider_table)

  provider_out = _tc_mlp(provider_table, prov_msg[:_N], prov_deg[:_N, None],
                         pW1, pb1, pg1, pbe1, pW2, pb2, pg2, pbe2)
  code_out = _tc_mlp(code_table, code_msg[:_N], code_deg[:_N, None],
                     cW1, cb1, cg1, cbe1, cW2, cb2, cg2, cbe2)
  return (provider_out, code_out)
